# baseline (speedup 1.0000x reference)
import functools

import numpy as np
import jax
import jax.numpy as jnp
from jax import lax
from jax.experimental import pallas as pl
from jax.experimental.pallas import tpu as pltpu
from jax.experimental.pallas import tpu_sc as plsc

_RB = 1024


def _tc_stats_body(x_ref, t_ref, lse_ref, rs_ref, xt_ref):
    x = x_ref[...]
    rb, v = x.shape
    m = jnp.max(x, axis=1)
    se = jnp.sum(jnp.exp(x - m[:, None]), axis=1)
    t = t_ref[0, 0, :]
    cols = lax.broadcasted_iota(jnp.int32, (rb, v), 1)
    xt = jnp.sum(jnp.where(cols == t[:, None], x, 0.0), axis=1)
    lse_ref[0, 0, :] = m + jnp.log(se)
    rs_ref[0, 0, :] = jnp.sum(x, axis=1)
    xt_ref[0, 0, :] = xt


def _tc_stats(x, t3):
    n, v = x.shape
    nblk = n // _RB
    vec = jax.ShapeDtypeStruct((nblk, 1, _RB), jnp.float32)
    return pl.pallas_call(
        _tc_stats_body,
        grid=(nblk,),
        in_specs=[
            pl.BlockSpec((_RB, v), lambda i: (i, 0)),
            pl.BlockSpec((1, 1, _RB), lambda i: (i, 0, 0)),
        ],
        out_specs=[
            pl.BlockSpec((1, 1, _RB), lambda i: (i, 0, 0)),
            pl.BlockSpec((1, 1, _RB), lambda i: (i, 0, 0)),
            pl.BlockSpec((1, 1, _RB), lambda i: (i, 0, 0)),
        ],
        out_shape=[vec, vec, vec],
    )(x, t3)


def _tc_combine_body(lse_ref, rs_ref, xt_ref, ns_ref, out_ref, *, c_smooth, v):
    lse = lse_ref[...]
    rs = rs_ref[...]
    xt = xt_ref[...]
    ns = ns_ref[...]
    w = ns * (c_smooth / (v - 1))
    src = 1.0 - v * w
    logp_t = xt - lse
    s_row = rs - v * lse
    ent = (v - 1.0) * (w * jnp.log(jnp.where(w > 0, w, 1.0))) \
        + src * jnp.log(jnp.where(src > 0, src, 1.0))
    cross = w * (s_row - logp_t) + src * logp_t
    out_ref[0, 0] = jnp.sum(ent - cross)


def _tc_combine(lse, rs, xt, ns3, c_smooth, v):
    return pl.pallas_call(
        functools.partial(_tc_combine_body, c_smooth=c_smooth, v=v),
        out_specs=pl.BlockSpec(memory_space=pltpu.SMEM),
        out_shape=jax.ShapeDtypeStruct((1, 1), jnp.float32),
    )(lse, rs, xt, ns3)


def _sc_fetch(table2d, row_idx, col_idx):
    info = plsc.get_sparse_core_info()
    nc, ns_sub = info.num_cores, info.num_subcores
    nw = nc * ns_sub
    n = row_idx.shape[0]
    bpw = n // nw
    mesh = plsc.VectorSubcoreMesh(core_axis_name="c", subcore_axis_name="s")

    r_base = row_idx >> 3
    c_tile = col_idx >> 7
    c_gran = (col_idx >> 4) & 7

    half = bpw // 2

    @functools.partial(
        pl.kernel,
        mesh=mesh,
        out_type=jax.ShapeDtypeStruct((n,), jnp.float32),
        scratch_types=[
            pltpu.VMEM((bpw,), jnp.int32),
            pltpu.VMEM((bpw,), jnp.int32),
            pltpu.VMEM((bpw,), jnp.int32),
            pltpu.VMEM((bpw,), jnp.int32),
            pltpu.VMEM((half * 8, 128), jnp.float32),
            pltpu.VMEM((bpw,), jnp.float32),
            pltpu.SemaphoreType.DMA,
        ],
    )
    def k(table_hbm, rb_hbm, ct_hbm, ro_hbm, co_hbm, out_hbm,
          rb_v, ct_v, ro_v, co_v, tiles_v, vals_v, sem):
        wid = lax.axis_index("s") * nc + lax.axis_index("c")
        base = wid * bpw
        pltpu.sync_copy(rb_hbm.at[pl.ds(base, bpw)], rb_v)
        pltpu.sync_copy(ct_hbm.at[pl.ds(base, bpw)], ct_v)
        pltpu.sync_copy(ro_hbm.at[pl.ds(base, bpw)], ro_v)
        pltpu.sync_copy(co_hbm.at[pl.ds(base, bpw)], co_v)
        lane = lax.iota(jnp.int32, 16)
        for h in range(bpw // half):
            descs = []
            for g in range(half // 16):
                rbv = rb_v[pl.ds(h * half + g * 16, 16)]
                ctv = ct_v[pl.ds(h * half + g * 16, 16)]
                for j in range(16):
                    d = pltpu.make_async_copy(
                        table_hbm.at[pl.ds(rbv[j] * 8, 8),
                                     pl.ds(ctv[j] * 128, 128)],
                        tiles_v.at[pl.ds((g * 16 + j) * 8, 8), :],
                        sem,
                    )
                    d.start()
                    descs.append(d)
            for d in descs:
                d.wait()
            for g in range(half // 16):
                o = h * half + g * 16
                rov = ro_v[pl.ds(o, 16)]
                cov = co_v[pl.ds(o, 16)]
                acc = jnp.zeros((16,), jnp.float32)
                for i in range(16):
                    ro = rov[i]
                    co = cov[i]
                    w = tiles_v[(g * 16 + i) * 8 + ro, pl.ds((co >> 4) * 16, 16)]
                    v16 = lax.gather(
                        w,
                        jnp.full((16, 1), co & 15, jnp.int32),
                        lax.GatherDimensionNumbers(
                            offset_dims=(), collapsed_slice_dims=(0,),
                            start_index_map=(0,)),
                        (1,),
                        mode=lax.GatherScatterMode.PROMISE_IN_BOUNDS,
                    )
                    acc = jnp.where(lane == i, v16, acc)
                vals_v[pl.ds(o, 16)] = acc
        pltpu.sync_copy(vals_v, out_hbm.at[pl.ds(base, bpw)])

    return k(table2d, r_base, c_tile, row_idx & 7, col_idx & 127)


def kernel(inputs, targets, placeholder, labels, matric):
    nb, nt = targets.shape
    n = nb * nt
    v = inputs.shape[-1]
    x = inputs.reshape(n, v)
    t = targets.reshape(-1)

    seg_count = labels.shape[0]
    seg_len = labels.shape[1] + 1
    c_smooth = float(1.0 - np.power(1.0 - 0.1, 1.0 / np.float64(seg_len)))

    prev = jnp.concatenate(
        [jnp.full((nb, 1), n - 1, dtype=targets.dtype), targets[:, : nt - 1]],
        axis=1,
    ).reshape(-1)
    need_smoothed = _sc_fetch(matric, prev, t)

    nblk = n // _RB
    t3 = t.reshape(nblk, 1, _RB)
    ns3 = need_smoothed.reshape(nblk, 1, _RB)
    lse, rs, xt = _tc_stats(x, t3)
    total = _tc_combine(lse, rs, xt, ns3, c_smooth, v)[0, 0]
    return total / jnp.float32(seg_count * seg_len)


# speedup vs baseline: 1.0130x; 1.0130x over previous
import functools

import numpy as np
import jax
import jax.numpy as jnp
from jax import lax
from jax.experimental import pallas as pl
from jax.experimental.pallas import tpu as pltpu
from jax.experimental.pallas import tpu_sc as plsc

_RB = 512


def _tc_stats_body(x_ref, t_ref, lse_ref, rs_ref, xt_ref):
    x = x_ref[...]
    rb, v = x.shape
    m = jnp.max(x, axis=1)
    se = jnp.sum(jnp.exp(x - m[:, None]), axis=1)
    t = t_ref[0, 0, :]
    cols = lax.broadcasted_iota(jnp.int32, (rb, v), 1)
    xt = jnp.sum(jnp.where(cols == t[:, None], x, 0.0), axis=1)
    lse_ref[0, 0, :] = m + jnp.log(se)
    rs_ref[0, 0, :] = jnp.sum(x, axis=1)
    xt_ref[0, 0, :] = xt


def _tc_stats(x, t3):
    n, v = x.shape
    nblk = n // _RB
    vec = jax.ShapeDtypeStruct((nblk, 1, _RB), jnp.float32)
    return pl.pallas_call(
        _tc_stats_body,
        grid=(nblk,),
        in_specs=[
            pl.BlockSpec((_RB, v), lambda i: (i, 0)),
            pl.BlockSpec((1, 1, _RB), lambda i: (i, 0, 0)),
        ],
        out_specs=[
            pl.BlockSpec((1, 1, _RB), lambda i: (i, 0, 0)),
            pl.BlockSpec((1, 1, _RB), lambda i: (i, 0, 0)),
            pl.BlockSpec((1, 1, _RB), lambda i: (i, 0, 0)),
        ],
        out_shape=[vec, vec, vec],
    )(x, t3)


def _tc_combine_body(lse_ref, rs_ref, xt_ref, ns_ref, out_ref, *, c_smooth, v):
    lse = lse_ref[...]
    rs = rs_ref[...]
    xt = xt_ref[...]
    ns = ns_ref[...]
    w = ns * (c_smooth / (v - 1))
    src = 1.0 - v * w
    logp_t = xt - lse
    s_row = rs - v * lse
    ent = (v - 1.0) * (w * jnp.log(jnp.where(w > 0, w, 1.0))) \
        + src * jnp.log(jnp.where(src > 0, src, 1.0))
    cross = w * (s_row - logp_t) + src * logp_t
    out_ref[0, 0] = jnp.sum(ent - cross)


def _tc_combine(lse, rs, xt, ns3, c_smooth, v):
    return pl.pallas_call(
        functools.partial(_tc_combine_body, c_smooth=c_smooth, v=v),
        out_specs=pl.BlockSpec(memory_space=pltpu.SMEM),
        out_shape=jax.ShapeDtypeStruct((1, 1), jnp.float32),
    )(lse, rs, xt, ns3)


def _sc_fetch(table2d, row_idx, col_idx):
    info = plsc.get_sparse_core_info()
    nc, ns_sub = info.num_cores, info.num_subcores
    nw = nc * ns_sub
    n = row_idx.shape[0]
    bpw = n // nw
    mesh = plsc.VectorSubcoreMesh(core_axis_name="c", subcore_axis_name="s")

    r_base = row_idx >> 3
    c_tile = col_idx >> 7
    c_gran = (col_idx >> 4) & 7

    half = bpw // 2

    @functools.partial(
        pl.kernel,
        mesh=mesh,
        out_type=jax.ShapeDtypeStruct((n,), jnp.float32),
        scratch_types=[
            pltpu.VMEM((bpw,), jnp.int32),
            pltpu.VMEM((bpw,), jnp.int32),
            pltpu.VMEM((bpw,), jnp.int32),
            pltpu.VMEM((bpw,), jnp.int32),
            pltpu.VMEM((half * 8, 128), jnp.float32),
            pltpu.VMEM((bpw,), jnp.float32),
            pltpu.SemaphoreType.DMA,
        ],
    )
    def k(table_hbm, rb_hbm, ct_hbm, ro_hbm, co_hbm, out_hbm,
          rb_v, ct_v, ro_v, co_v, tiles_v, vals_v, sem):
        wid = lax.axis_index("s") * nc + lax.axis_index("c")
        base = wid * bpw
        pltpu.sync_copy(rb_hbm.at[pl.ds(base, bpw)], rb_v)
        pltpu.sync_copy(ct_hbm.at[pl.ds(base, bpw)], ct_v)
        pltpu.sync_copy(ro_hbm.at[pl.ds(base, bpw)], ro_v)
        pltpu.sync_copy(co_hbm.at[pl.ds(base, bpw)], co_v)
        lane = lax.iota(jnp.int32, 16)
        for h in range(bpw // half):
            descs = []
            for g in range(half // 16):
                rbv = rb_v[pl.ds(h * half + g * 16, 16)]
                ctv = ct_v[pl.ds(h * half + g * 16, 16)]
                for j in range(16):
                    d = pltpu.make_async_copy(
                        table_hbm.at[pl.ds(rbv[j] * 8, 8),
                                     pl.ds(ctv[j] * 128, 128)],
                        tiles_v.at[pl.ds((g * 16 + j) * 8, 8), :],
                        sem,
                    )
                    d.start()
                    descs.append(d)
            for d in descs:
                d.wait()
            for g in range(half // 16):
                o = h * half + g * 16
                rov = ro_v[pl.ds(o, 16)]
                cov = co_v[pl.ds(o, 16)]
                acc = jnp.zeros((16,), jnp.float32)
                for i in range(16):
                    ro = rov[i]
                    co = cov[i]
                    w = tiles_v[(g * 16 + i) * 8 + ro, pl.ds((co >> 4) * 16, 16)]
                    v16 = lax.gather(
                        w,
                        jnp.full((16, 1), co & 15, jnp.int32),
                        lax.GatherDimensionNumbers(
                            offset_dims=(), collapsed_slice_dims=(0,),
                            start_index_map=(0,)),
                        (1,),
                        mode=lax.GatherScatterMode.PROMISE_IN_BOUNDS,
                    )
                    acc = jnp.where(lane == i, v16, acc)
                vals_v[pl.ds(o, 16)] = acc
        pltpu.sync_copy(vals_v, out_hbm.at[pl.ds(base, bpw)])

    return k(table2d, r_base, c_tile, row_idx & 7, col_idx & 127)


def kernel(inputs, targets, placeholder, labels, matric):
    nb, nt = targets.shape
    n = nb * nt
    v = inputs.shape[-1]
    x = inputs.reshape(n, v)
    t = targets.reshape(-1)

    seg_count = labels.shape[0]
    seg_len = labels.shape[1] + 1
    c_smooth = float(1.0 - np.power(1.0 - 0.1, 1.0 / np.float64(seg_len)))

    prev = jnp.concatenate(
        [jnp.full((nb, 1), n - 1, dtype=targets.dtype), targets[:, : nt - 1]],
        axis=1,
    ).reshape(-1)
    need_smoothed = _sc_fetch(matric, prev, t)

    nblk = n // _RB
    t3 = t.reshape(nblk, 1, _RB)
    ns3 = need_smoothed.reshape(nblk, 1, _RB)
    lse, rs, xt = _tc_stats(x, t3)
    total = _tc_combine(lse, rs, xt, ns3, c_smooth, v)[0, 0]
    return total / jnp.float32(seg_count * seg_len)


# speedup vs baseline: 1.0983x; 1.0842x over previous
import functools

import numpy as np
import jax
import jax.numpy as jnp
from jax import lax
from jax.experimental import pallas as pl
from jax.experimental.pallas import tpu as pltpu
from jax.experimental.pallas import tpu_sc as plsc

_RB = 512


def _tc_stats_body(x_ref, t_ref, lse_ref, rs_ref, xt_ref):
    x = x_ref[...]
    rb, v = x.shape
    m = jnp.max(x, axis=1)
    se = jnp.sum(jnp.exp(x - m[:, None]), axis=1)
    t = t_ref[0, 0, :]
    cols = lax.broadcasted_iota(jnp.int32, (rb, v), 1)
    xt = jnp.sum(jnp.where(cols == t[:, None], x, 0.0), axis=1)
    lse_ref[0, 0, :] = m + jnp.log(se)
    rs_ref[0, 0, :] = jnp.sum(x, axis=1)
    xt_ref[0, 0, :] = xt


def _tc_stats(x, t3):
    n, v = x.shape
    nblk = n // _RB
    vec = jax.ShapeDtypeStruct((nblk, 1, _RB), jnp.float32)
    return pl.pallas_call(
        _tc_stats_body,
        grid=(nblk,),
        in_specs=[
            pl.BlockSpec((_RB, v), lambda i: (i, 0)),
            pl.BlockSpec((1, 1, _RB), lambda i: (i, 0, 0)),
        ],
        out_specs=[
            pl.BlockSpec((1, 1, _RB), lambda i: (i, 0, 0)),
            pl.BlockSpec((1, 1, _RB), lambda i: (i, 0, 0)),
            pl.BlockSpec((1, 1, _RB), lambda i: (i, 0, 0)),
        ],
        out_shape=[vec, vec, vec],
    )(x, t3)


def _tc_combine_body(lse_ref, rs_ref, xt_ref, ns_ref, out_ref, *, c_smooth, v):
    lse = lse_ref[...]
    rs = rs_ref[...]
    xt = xt_ref[...]
    ns = ns_ref[...]
    w = ns * (c_smooth / (v - 1))
    src = 1.0 - v * w
    logp_t = xt - lse
    s_row = rs - v * lse
    ent = (v - 1.0) * (w * jnp.log(jnp.where(w > 0, w, 1.0))) \
        + src * jnp.log(jnp.where(src > 0, src, 1.0))
    cross = w * (s_row - logp_t) + src * logp_t
    out_ref[0, 0] = jnp.sum(ent - cross)


def _tc_combine(lse, rs, xt, ns3, c_smooth, v):
    return pl.pallas_call(
        functools.partial(_tc_combine_body, c_smooth=c_smooth, v=v),
        out_specs=pl.BlockSpec(memory_space=pltpu.SMEM),
        out_shape=jax.ShapeDtypeStruct((1, 1), jnp.float32),
    )(lse, rs, xt, ns3)


def _sc_fetch(table2d, row_idx, col_idx):
    info = plsc.get_sparse_core_info()
    nc, ns_sub = info.num_cores, info.num_subcores
    nw = nc * ns_sub
    n = row_idx.shape[0]
    bpw = n // nw
    mesh = plsc.VectorSubcoreMesh(core_axis_name="c", subcore_axis_name="s")

    nrow, ncol = table2d.shape
    view = (
        table2d.reshape(nrow // 8, 8, ncol // 128, 128)
        .swapaxes(1, 2)
        .reshape(nrow * ncol // 128, 128)
    )
    vrow = ((row_idx >> 3) * (ncol // 128) + (col_idx >> 7)) * 8 + (row_idx & 7)
    vlane = col_idx & 127

    @functools.partial(
        pl.kernel,
        mesh=mesh,
        out_type=jax.ShapeDtypeStruct((n,), jnp.float32),
        scratch_types=[
            pltpu.VMEM((bpw,), jnp.int32),
            pltpu.VMEM((bpw,), jnp.int32),
            pltpu.VMEM((bpw, 128), jnp.float32),
            pltpu.VMEM((bpw,), jnp.float32),
            pltpu.SemaphoreType.DMA,
        ],
    )
    def k(view_hbm, vr_hbm, ln_hbm, out_hbm, vr_v, ln_v, rows_v, vals_v, sem):
        wid = lax.axis_index("s") * nc + lax.axis_index("c")
        base = wid * bpw
        pltpu.sync_copy(vr_hbm.at[pl.ds(base, bpw)], vr_v)
        pltpu.sync_copy(ln_hbm.at[pl.ds(base, bpw)], ln_v)
        pltpu.async_copy(view_hbm.at[vr_v], rows_v, sem).wait()
        lane = lax.iota(jnp.int32, 16)
        for g in range(bpw // 16):
            o = g * 16
            lnv = ln_v[pl.ds(o, 16)]
            acc = jnp.zeros((16,), jnp.float32)
            for i in range(16):
                ln = lnv[i]
                w = rows_v[o + i, pl.ds((ln >> 4) * 16, 16)]
                v16 = lax.gather(
                    w,
                    jnp.full((16, 1), ln & 15, jnp.int32),
                    lax.GatherDimensionNumbers(
                        offset_dims=(), collapsed_slice_dims=(0,),
                        start_index_map=(0,)),
                    (1,),
                    mode=lax.GatherScatterMode.PROMISE_IN_BOUNDS,
                )
                acc = jnp.where(lane == i, v16, acc)
            vals_v[pl.ds(o, 16)] = acc
        pltpu.sync_copy(vals_v, out_hbm.at[pl.ds(base, bpw)])

    return k(view, vrow, vlane)


def kernel(inputs, targets, placeholder, labels, matric):
    nb, nt = targets.shape
    n = nb * nt
    v = inputs.shape[-1]
    x = inputs.reshape(n, v)
    t = targets.reshape(-1)

    seg_count = labels.shape[0]
    seg_len = labels.shape[1] + 1
    c_smooth = float(1.0 - np.power(1.0 - 0.1, 1.0 / np.float64(seg_len)))

    prev = jnp.concatenate(
        [jnp.full((nb, 1), n - 1, dtype=targets.dtype), targets[:, : nt - 1]],
        axis=1,
    ).reshape(-1)
    need_smoothed = _sc_fetch(matric, prev, t)

    nblk = n // _RB
    t3 = t.reshape(nblk, 1, _RB)
    ns3 = need_smoothed.reshape(nblk, 1, _RB)
    lse, rs, xt = _tc_stats(x, t3)
    total = _tc_combine(lse, rs, xt, ns3, c_smooth, v)[0, 0]
    return total / jnp.float32(seg_count * seg_len)


# speedup vs baseline: 1.1008x; 1.0023x over previous
import functools

import numpy as np
import jax
import jax.numpy as jnp
from jax import lax
from jax.experimental import pallas as pl
from jax.experimental.pallas import tpu as pltpu
from jax.experimental.pallas import tpu_sc as plsc

_RB = 512


def _tc_stats_body(x_ref, t_ref, lse_ref, rs_ref, xt_ref):
    x = x_ref[...]
    rb, v = x.shape
    m = jnp.max(x, axis=1)
    se = jnp.sum(jnp.exp(x - m[:, None]), axis=1)
    t = t_ref[0, 0, :]
    cols = lax.broadcasted_iota(jnp.int32, (rb, v), 1)
    xt = jnp.sum(jnp.where(cols == t[:, None], x, 0.0), axis=1)
    lse_ref[0, 0, :] = m + jnp.log(se)
    rs_ref[0, 0, :] = jnp.sum(x, axis=1)
    xt_ref[0, 0, :] = xt


def _tc_stats(x, t3):
    n, v = x.shape
    nblk = n // _RB
    vec = jax.ShapeDtypeStruct((nblk, 1, _RB), jnp.float32)
    return pl.pallas_call(
        _tc_stats_body,
        grid=(nblk,),
        in_specs=[
            pl.BlockSpec((_RB, v), lambda i: (i, 0)),
            pl.BlockSpec((1, 1, _RB), lambda i: (i, 0, 0)),
        ],
        out_specs=[
            pl.BlockSpec((1, 1, _RB), lambda i: (i, 0, 0)),
            pl.BlockSpec((1, 1, _RB), lambda i: (i, 0, 0)),
            pl.BlockSpec((1, 1, _RB), lambda i: (i, 0, 0)),
        ],
        out_shape=[vec, vec, vec],
    )(x, t3)


def _tc_combine_body(lse_ref, rs_ref, xt_ref, ns_ref, out_ref, *, c_smooth, v):
    lse = lse_ref[...]
    rs = rs_ref[...]
    xt = xt_ref[...]
    ns = ns_ref[...]
    w = ns * (c_smooth / (v - 1))
    src = 1.0 - v * w
    logp_t = xt - lse
    s_row = rs - v * lse
    ent = (v - 1.0) * (w * jnp.log(jnp.where(w > 0, w, 1.0))) \
        + src * jnp.log(jnp.where(src > 0, src, 1.0))
    cross = w * (s_row - logp_t) + src * logp_t
    out_ref[0, 0] = jnp.sum(ent - cross)


def _tc_combine(lse, rs, xt, ns3, c_smooth, v):
    return pl.pallas_call(
        functools.partial(_tc_combine_body, c_smooth=c_smooth, v=v),
        out_specs=pl.BlockSpec(memory_space=pltpu.SMEM),
        out_shape=jax.ShapeDtypeStruct((1, 1), jnp.float32),
    )(lse, rs, xt, ns3)


def _sc_fetch(table2d, row_idx, col_idx):
    info = plsc.get_sparse_core_info()
    nc, ns_sub = info.num_cores, info.num_subcores
    nw = nc * ns_sub
    n = row_idx.shape[0]
    bpw = n // nw
    mesh = plsc.VectorSubcoreMesh(core_axis_name="c", subcore_axis_name="s")

    nrow, ncol = table2d.shape
    view = (
        table2d.reshape(nrow // 8, 8, ncol // 128, 128)
        .swapaxes(1, 2)
        .reshape(nrow * ncol // 128, 128)
    )
    vrow = ((row_idx >> 3) * (ncol // 128) + (col_idx >> 7)) * 8 + (row_idx & 7)
    vlane = col_idx & 127

    @functools.partial(
        pl.kernel,
        mesh=mesh,
        out_type=jax.ShapeDtypeStruct((n,), jnp.float32),
        scratch_types=[
            pltpu.VMEM((bpw,), jnp.int32),
            pltpu.VMEM((bpw,), jnp.int32),
            pltpu.VMEM((bpw, 128), jnp.float32),
            pltpu.VMEM((bpw,), jnp.float32),
            pltpu.SemaphoreType.DMA,
        ],
    )
    def k(view_hbm, vr_hbm, ln_hbm, out_hbm, vr_v, ln_v, rows_v, vals_v, sem):
        wid = lax.axis_index("s") * nc + lax.axis_index("c")
        base = wid * bpw
        pltpu.sync_copy(vr_hbm.at[pl.ds(base, bpw)], vr_v)
        pltpu.sync_copy(ln_hbm.at[pl.ds(base, bpw)], ln_v)
        pltpu.async_copy(view_hbm.at[vr_v], rows_v, sem).wait()
        lane = lax.iota(jnp.int32, 16)
        for g in range(bpw // 16):
            o = g * 16
            lnv = ln_v[pl.ds(o, 16)]
            acc = jnp.zeros((16,), jnp.float32)
            for i in range(16):
                ln = lnv[i]
                w = rows_v[o + i, pl.ds((ln >> 4) * 16, 16)]
                v16 = lax.gather(
                    w,
                    jnp.full((16, 1), ln & 15, jnp.int32),
                    lax.GatherDimensionNumbers(
                        offset_dims=(), collapsed_slice_dims=(0,),
                        start_index_map=(0,)),
                    (1,),
                    mode=lax.GatherScatterMode.PROMISE_IN_BOUNDS,
                )
                acc = jnp.where(lane == i, v16, acc)
            vals_v[pl.ds(o, 16)] = acc
        pltpu.sync_copy(vals_v, out_hbm.at[pl.ds(base, bpw)])

    return k(view, vrow, vlane)


def kernel(inputs, targets, placeholder, labels, matric):
    nb, nt = targets.shape
    n = nb * nt
    v = inputs.shape[-1]
    x = inputs.reshape(n, v)
    t = targets.reshape(-1)

    seg_count = labels.shape[0]
    seg_len = labels.shape[1] + 1
    c_smooth = float(1.0 - np.power(1.0 - 0.1, 1.0 / np.float64(seg_len)))

    prev = jnp.concatenate(
        [jnp.full((nb, 1), n - 1, dtype=targets.dtype), targets[:, : nt - 1]],
        axis=1,
    ).reshape(-1)
    nblk = n // _RB
    t3 = t.reshape(nblk, 1, _RB)
    lse, rs, xt = _tc_stats(x, t3)
    need_smoothed = _sc_fetch(matric, prev, t)
    ns3 = need_smoothed.reshape(nblk, 1, _RB)
    total = _tc_combine(lse, rs, xt, ns3, c_smooth, v)[0, 0]
    return total / jnp.float32(seg_count * seg_len)
